# baseline (speedup 1.0000x reference)
import functools

import jax
import jax.numpy as jnp
from jax import lax
from jax.experimental import pallas as pl
from jax.experimental.pallas import tpu as pltpu
from jax.experimental.pallas import tpu_sc as plsc

N = 10000
E = 160000
HID = 128
NC, NS = 2, 16
NT = NC * NS
EPAD = 163840
EPT = EPAD // NT
KE = 128
NCH = EPT // KE
KG = 64
NCHG = EPT // KG
NB = 4
ACC_ROWS = 10112
RPT = ACC_ROWS // NS
XPAD = 10240
KX = 64
NCHX = (XPAD // NT) // KX

BN_E = 1024
BN_N = 1000


def _vmesh():
    return plsc.VectorSubcoreMesh(
        core_axis_name="c", subcore_axis_name="s",
        num_cores=NC, num_subcores=NS)


def _sc_gather1(table, idxc):
    nch, k = idxc.shape[1], idxc.shape[2]
    d = table.shape[1]
    per_tile = nch * k

    @functools.partial(
        pl.kernel,
        out_type=jax.ShapeDtypeStruct((NT * per_tile, d), jnp.float32),
        mesh=_vmesh(),
        scratch_types=[
            pltpu.VMEM((nch, k), jnp.int32),
            pltpu.VMEM((k, d), jnp.float32),
            pltpu.SemaphoreType.DMA,
        ],
    )
    def k_fn(tab_hbm, idx_hbm, out_hbm, idx_v, buf, sem):
        wid = lax.axis_index("s") * NC + lax.axis_index("c")
        base = wid * per_tile
        pltpu.sync_copy(idx_hbm.at[wid], idx_v)

        def body(c, carry):
            pltpu.async_copy(tab_hbm.at[idx_v.at[c]], buf, sem).wait()
            pltpu.sync_copy(buf, out_hbm.at[pl.ds(base + c * k, k)])
            return carry

        lax.fori_loop(0, nch, body, 0)

    return k_fn(table, idxc)


def _sc_gather2(p_tab, q_tab, dstc, srcc):
    d = p_tab.shape[1]
    dt = p_tab.dtype

    @functools.partial(
        pl.kernel,
        out_type=(jax.ShapeDtypeStruct((EPAD, d), dt),
                  jax.ShapeDtypeStruct((EPAD, d), dt)),
        mesh=_vmesh(),
        scratch_types=(
            [pltpu.VMEM((NCHG, KG), jnp.int32),
             pltpu.VMEM((NCHG, KG), jnp.int32)]
            + [pltpu.VMEM((KG, d), dt) for _ in range(2 * NB)]
            + [pltpu.SemaphoreType.DMA for _ in range(4 * NB)]
        ),
    )
    def k_fn(p_hbm, q_hbm, dst_hbm, src_hbm, pd_hbm, qs_hbm,
             idxd, idxs, *rest):
        buf_p = rest[0:NB]
        buf_q = rest[NB:2 * NB]
        gsem_p = rest[2 * NB:3 * NB]
        gsem_q = rest[3 * NB:4 * NB]
        wsem_p = rest[4 * NB:5 * NB]
        wsem_q = rest[5 * NB:6 * NB]
        wid = lax.axis_index("s") * NC + lax.axis_index("c")
        base = wid * EPT
        pltpu.sync_copy(dst_hbm.at[wid], idxd)
        pltpu.sync_copy(src_hbm.at[wid], idxs)

        def body(i, carry):
            hs = []
            for b in range(NB):
                c = i * NB + b

                @pl.when(i > 0)
                def _(b=b):
                    pltpu.make_async_copy(
                        pd_hbm.at[pl.ds(0, KG)], buf_p[b], wsem_p[b]).wait()
                    pltpu.make_async_copy(
                        qs_hbm.at[pl.ds(0, KG)], buf_q[b], wsem_q[b]).wait()

                hp = pltpu.async_copy(p_hbm.at[idxd.at[c]], buf_p[b],
                                      gsem_p[b])
                hq = pltpu.async_copy(q_hbm.at[idxs.at[c]], buf_q[b],
                                      gsem_q[b])
                hs.append((hp, hq))
            for b in range(NB):
                c = i * NB + b
                hp, hq = hs[b]
                hp.wait()
                pltpu.async_copy(buf_p[b],
                                 pd_hbm.at[pl.ds(base + c * KG, KG)],
                                 wsem_p[b])
                hq.wait()
                pltpu.async_copy(buf_q[b],
                                 qs_hbm.at[pl.ds(base + c * KG, KG)],
                                 wsem_q[b])
            return carry

        lax.fori_loop(0, NCHG // NB, body, 0)
        for b in range(NB):
            pltpu.make_async_copy(
                pd_hbm.at[pl.ds(0, KG)], buf_p[b], wsem_p[b]).wait()
            pltpu.make_async_copy(
                qs_hbm.at[pl.ds(0, KG)], buf_q[b], wsem_q[b]).wait()

    return k_fn(p_tab, q_tab, dstc, srcc)


def _sc_scatter(m, dstc, zrows):
    NBS = 2

    @functools.partial(
        pl.kernel,
        out_type=jax.ShapeDtypeStruct((NC, ACC_ROWS, HID), jnp.float32),
        mesh=_vmesh(),
        scratch_types=(
            [pltpu.VMEM((NCH, KE), jnp.int32)]
            + [pltpu.VMEM((KE, HID), jnp.float32) for _ in range(NBS)]
            + [pltpu.VMEM_SHARED((ACC_ROWS, HID), jnp.float32)]
            + [pltpu.SemaphoreType.DMA for _ in range(2 * NBS)]
        ),
    )
    def k_fn(m_hbm, dst_hbm, z_hbm, s_hbm, idxd, *rest):
        bufs = rest[0:NBS]
        acc = rest[NBS]
        rsem = rest[NBS + 1:2 * NBS + 1]
        asem = rest[2 * NBS + 1:3 * NBS + 1]
        cid = lax.axis_index("c")
        sid = lax.axis_index("s")
        wid = sid * NC + cid
        base = wid * EPT
        pltpu.sync_copy(z_hbm, acc.at[pl.ds(sid * RPT, RPT)])
        pltpu.sync_copy(dst_hbm.at[wid], idxd)
        plsc.subcore_barrier()

        def body(i, carry):
            hs = []
            for b in range(NBS):
                c = i * NBS + b

                @pl.when(i > 0)
                def _(b=b):
                    pltpu.make_async_copy(
                        m_hbm.at[pl.ds(0, KE)], bufs[b], asem[b]).wait()

                hs.append(pltpu.async_copy(
                    m_hbm.at[pl.ds(base + c * KE, KE)], bufs[b], rsem[b]))
            for b in range(NBS):
                c = i * NBS + b
                hs[b].wait()
                pltpu.async_copy(bufs[b], acc.at[idxd.at[c]], asem[b],
                                 add=True)
            return carry

        lax.fori_loop(0, NCH // NBS, body, 0)
        for b in range(NBS):
            pltpu.make_async_copy(
                m_hbm.at[pl.ds(0, KE)], bufs[b], asem[b]).wait()
        plsc.subcore_barrier()
        pltpu.sync_copy(acc.at[pl.ds(sid * RPT, RPT)],
                        s_hbm.at[cid, pl.ds(sid * RPT, RPT)])

    return k_fn(m, dstc, zrows)


def _layernorm(a, g, b):
    mu = jnp.mean(a, axis=-1, keepdims=True)
    var = jnp.mean((a - mu) ** 2, axis=-1, keepdims=True)
    return (a - mu) * lax.rsqrt(var + 1e-5) * g + b


def _dot(a, b, preferred_element_type=jnp.float32):
    return jnp.dot(a, b, preferred_element_type=preferred_element_type)


def _pack_bf16(pv):
    d2 = pv.shape[-1] // 2
    lo = pv[:, :d2].astype(jnp.bfloat16).astype(jnp.float32)
    hi = pv[:, d2:].astype(jnp.bfloat16).astype(jnp.float32)
    lo_u = lax.bitcast_convert_type(lo, jnp.uint32)
    hi_u = lax.bitcast_convert_type(hi, jnp.uint32)
    return lax.bitcast_convert_type(hi_u | (lo_u >> 16), jnp.int32)


def _unpack_bf16(w):
    u = lax.bitcast_convert_type(w, jnp.uint32)
    lo = lax.bitcast_convert_type(u << 16, jnp.float32)
    hi = lax.bitcast_convert_type(u & jnp.uint32(0xFFFF0000), jnp.float32)
    return lo, hi

def _row_spec(bn, d):
    return pl.BlockSpec((bn, d), lambda i: (i, 0))


def _w_spec(shape):
    if len(shape) == 1:
        return pl.BlockSpec(shape, lambda i: (0,))
    return pl.BlockSpec(shape, lambda i: (0, 0))


def _tc_embed_nodes(h0, xf, p):
    wl, bl = p["embed_x"]["lin"]["w"], p["embed_x"]["lin"]["b"]
    gx, bx = p["embed_x"]["ln"]["g"], p["embed_x"]["ln"]["b"]
    pf = p["embed_feat"]
    wf1, bf1 = pf["l1"]["w"], pf["l1"]["b"]
    wf2, bf2 = pf["l2"]["w"], pf["l2"]["b"]
    gf, bf = pf["ln"]["g"], pf["ln"]["b"]
    df = xf.shape[1]

    def body(h0_ref, xf_ref, wl_ref, bl_ref, gx_ref, bx_ref,
             wf1_ref, bf1_ref, wf2_ref, bf2_ref, gf_ref, bf_ref,
             hres_ref, pack_ref):
        h = jnp.maximum(h0_ref[...], 0.0)
        h = _dot(h, wl_ref[...], preferred_element_type=jnp.float32) + bl_ref[...]
        h = _layernorm(h, gx_ref[...], bx_ref[...])
        f = _dot(xf_ref[...], wf1_ref[...], preferred_element_type=jnp.float32) + bf1_ref[...]
        f = jnp.maximum(f, 0.0)
        f = _dot(f, wf2_ref[...], preferred_element_type=jnp.float32) + bf2_ref[...]
        f = _layernorm(f, gf_ref[...], bf_ref[...])
        hres_ref[...] = h
        pack_ref[...] = jnp.concatenate([h, f], axis=-1)

    return pl.pallas_call(
        body,
        grid=(N // BN_N,),
        in_specs=[_row_spec(BN_N, HID), _row_spec(BN_N, df),
                  _w_spec(wl.shape), _w_spec(bl.shape),
                  _w_spec(gx.shape), _w_spec(bx.shape),
                  _w_spec(wf1.shape), _w_spec(bf1.shape),
                  _w_spec(wf2.shape), _w_spec(bf2.shape),
                  _w_spec(gf.shape), _w_spec(bf.shape)],
        out_specs=(_row_spec(BN_N, HID), _row_spec(BN_N, 2 * HID)),
        out_shape=(jax.ShapeDtypeStruct((N, HID), jnp.float32),
                   jax.ShapeDtypeStruct((N, 2 * HID), jnp.float32)),
    )(h0, xf, wl, bl, gx, bx, wf1, bf1, wf2, bf2, gf, bf)


def _tc_embed_edges(ea, p):
    w1, b1 = p["l1"]["w"], p["l1"]["b"]
    w2, b2 = p["l2"]["w"], p["l2"]["b"]
    g, b = p["ln"]["g"], p["ln"]["b"]
    de = ea.shape[1]

    def body(ea_ref, w1_ref, b1_ref, w2_ref, b2_ref, g_ref, b_ref, out_ref):
        a = _dot(ea_ref[...], w1_ref[...], preferred_element_type=jnp.float32) + b1_ref[...]
        a = jnp.maximum(a, 0.0)
        a = _dot(a, w2_ref[...], preferred_element_type=jnp.float32) + b2_ref[...]
        out_ref[...] = _layernorm(a, g_ref[...], b_ref[...])

    return pl.pallas_call(
        body,
        grid=(EPAD // BN_E,),
        in_specs=[_row_spec(BN_E, de),
                  _w_spec(w1.shape), _w_spec(b1.shape),
                  _w_spec(w2.shape), _w_spec(b2.shape),
                  _w_spec(g.shape), _w_spec(b.shape)],
        out_specs=_row_spec(BN_E, HID),
        out_shape=jax.ShapeDtypeStruct((EPAD, HID), jnp.float32),
    )(ea, w1, b1, w2, b2, g, b)


def _tc_pq(h, wd, ws, relu_h):
    din = h.shape[1]

    def body(h_ref, wd_ref, ws_ref, p_ref, q_ref):
        hv = h_ref[...]
        if relu_h:
            hv = jnp.maximum(hv, 0.0)
        p_ref[...] = _pack_bf16(_dot(hv, wd_ref[...]))
        q_ref[...] = _pack_bf16(_dot(hv, ws_ref[...]))

    return pl.pallas_call(
        body,
        grid=(N // BN_N,),
        in_specs=[_row_spec(BN_N, din), _w_spec(wd.shape), _w_spec(ws.shape)],
        out_specs=(_row_spec(BN_N, HID), _row_spec(BN_N, HID)),
        out_shape=(jax.ShapeDtypeStruct((N, HID), jnp.int32),
                   jax.ShapeDtypeStruct((N, HID), jnp.int32)),
    )(h, wd, ws)


def _tc_edge_mlp(pd, qs, e, wc, b1, w2, b2, bn_e, relu_e):
    se = bn_e["g"] * (1.0 / jnp.sqrt(1.0 + 1e-5))
    be = bn_e["b"]

    def body(pd_ref, qs_ref, e_ref, wc_ref, b1_ref, w2_ref, b2_ref,
             se_ref, be_ref, m_ref, eo_ref):
        er = e_ref[...]
        if relu_e:
            er = jnp.maximum(er, 0.0)
        pd_lo, pd_hi = _unpack_bf16(pd_ref[...])
        qs_lo, qs_hi = _unpack_bf16(qs_ref[...])
        c = _dot(er, wc_ref[...], preferred_element_type=jnp.float32)
        b1v = b1_ref[...]
        z_lo = jnp.maximum(pd_lo + qs_lo + c[:, :HID] + b1v[:HID], 0.0)
        z_hi = jnp.maximum(pd_hi + qs_hi + c[:, HID:] + b1v[HID:], 0.0)
        m = (_dot(z_lo, w2_ref[:HID, :], preferred_element_type=jnp.float32)
             + _dot(z_hi, w2_ref[HID:, :], preferred_element_type=jnp.float32)
             + b2_ref[...])
        m_ref[...] = m
        eo_ref[...] = er + m * se_ref[...] + be_ref[...]

    return pl.pallas_call(
        body,
        grid=(EPAD // BN_E,),
        in_specs=[_row_spec(BN_E, HID), _row_spec(BN_E, HID),
                  _row_spec(BN_E, HID),
                  _w_spec(wc.shape), _w_spec(b1.shape),
                  _w_spec(w2.shape), _w_spec(b2.shape),
                  _w_spec(se.shape), _w_spec(be.shape)],
        out_specs=(_row_spec(BN_E, HID), _row_spec(BN_E, HID)),
        out_shape=(jax.ShapeDtypeStruct((EPAD, HID), jnp.float32),
                   jax.ShapeDtypeStruct((EPAD, HID), jnp.float32)),
    )(pd, qs, e, wc, b1, w2, b2, se, be)


def _tc_update(h_prev, s, bn_x, relu_h, lin=None):
    sx = bn_x["g"] * (1.0 / jnp.sqrt(1.0 + 1e-5))
    bx = bn_x["b"]
    args = [h_prev, s, sx, bx]
    in_specs = [_row_spec(BN_N, HID),
                pl.BlockSpec((NC, BN_N, HID), lambda i: (0, i, 0)),
                _w_spec(sx.shape), _w_spec(bx.shape)]
    if lin is not None:
        args += [lin["w"], lin["b"]]
        in_specs += [_w_spec(lin["w"].shape), _w_spec(lin["b"].shape)]

    def body(h_ref, s_ref, sx_ref, bx_ref, *rest):
        hv = h_ref[...]
        if relu_h:
            hv = jnp.maximum(hv, 0.0)
        ssum = s_ref[0, :, :] + s_ref[1, :, :]
        hs = hv + ssum * sx_ref[...] + bx_ref[...]
        if lin is not None:
            w_ref, b_ref, out_ref = rest
            out_ref[...] = _dot(hs, w_ref[...],
                                preferred_element_type=jnp.float32) + b_ref[...]
        else:
            out_ref, = rest
            out_ref[...] = hs

    dout = HID if lin is None else lin["w"].shape[1]
    return pl.pallas_call(
        body,
        grid=(N // BN_N,),
        in_specs=in_specs,
        out_specs=_row_spec(BN_N, dout),
        out_shape=jax.ShapeDtypeStruct((N, dout), jnp.float32),
    )(*args)


def kernel(x, edge_index, edge_attr, x_feat, params):
    x = x.astype(jnp.int32)
    src = edge_index[0].astype(jnp.int32)
    dst = edge_index[1].astype(jnp.int32)

    dst_g = jnp.pad(dst, (0, EPAD - E)).reshape(NT, NCHG, KG)
    src_g = jnp.pad(src, (0, EPAD - E)).reshape(NT, NCHG, KG)
    dst_s = jnp.pad(dst, (0, EPAD - E), constant_values=N).reshape(NT, NCH, KE)
    xc = jnp.pad(x, (0, XPAD - N)).reshape(NT, NCHX, KX)
    ea = jnp.pad(edge_attr, ((0, EPAD - E), (0, 0)))
    zrows = jnp.zeros((RPT, HID), jnp.float32)

    p = params
    h0 = _sc_gather1(p["embed_x"]["table"], xc)[:N]
    h_res, hcat = _tc_embed_nodes(h0, x_feat, p)
    e = _tc_embed_edges(ea, p["embed_adj"])

    h_in, relu_h = hcat, False
    out = None
    for li in range(4):
        cp = p["conv0"] if li == 0 else p["convs"][li - 1]
        d = h_in.shape[1]
        w1 = cp["m1"]["w"]
        wd = w1[:d] - w1[d:2 * d]
        ws = w1[d:2 * d]
        wc = w1[2 * d:]
        p_tab, q_tab = _tc_pq(h_in, wd, ws, relu_h)
        pd, qs = _sc_gather2(p_tab, q_tab, dst_g, src_g)
        m, e = _tc_edge_mlp(pd, qs, e, wc, cp["m1"]["b"], cp["m2"]["w"],
                            cp["m2"]["b"], cp["bn_e"], relu_e=(li > 0))
        s = _sc_scatter(m, dst_s, zrows)
        if li < 3:
            h_res = _tc_update(h_res, s, cp["bn_x"], relu_h=(li > 0))
            h_in, relu_h = h_res, True
        else:
            out = _tc_update(h_res, s, cp["bn_x"], relu_h=True,
                             lin=p["lin_out"])
    return out


# speedup vs baseline: 1.0529x; 1.0529x over previous
import functools

import jax
import jax.numpy as jnp
from jax import lax
from jax.experimental import pallas as pl
from jax.experimental.pallas import tpu as pltpu
from jax.experimental.pallas import tpu_sc as plsc

N = 10000
E = 160000
HID = 128
NC, NS = 2, 16
NT = NC * NS
EPAD = 163840
EPT = EPAD // NT
KE = 128
NCH = EPT // KE
KG = 128
NCHG = EPT // KG
NB = 2
ACC_ROWS = 10112
RPT = ACC_ROWS // NS
XPAD = 10240
KX = 64
NCHX = (XPAD // NT) // KX

BN_E = 1024
BN_N = 1000


def _vmesh():
    return plsc.VectorSubcoreMesh(
        core_axis_name="c", subcore_axis_name="s",
        num_cores=NC, num_subcores=NS)


def _sc_gather1(table, idxc):
    nch, k = idxc.shape[1], idxc.shape[2]
    d = table.shape[1]
    per_tile = nch * k

    @functools.partial(
        pl.kernel,
        out_type=jax.ShapeDtypeStruct((NT * per_tile, d), jnp.float32),
        mesh=_vmesh(),
        scratch_types=[
            pltpu.VMEM((nch, k), jnp.int32),
            pltpu.VMEM((k, d), jnp.float32),
            pltpu.SemaphoreType.DMA,
        ],
    )
    def k_fn(tab_hbm, idx_hbm, out_hbm, idx_v, buf, sem):
        wid = lax.axis_index("s") * NC + lax.axis_index("c")
        base = wid * per_tile
        pltpu.sync_copy(idx_hbm.at[wid], idx_v)

        def body(c, carry):
            pltpu.async_copy(tab_hbm.at[idx_v.at[c]], buf, sem).wait()
            pltpu.sync_copy(buf, out_hbm.at[pl.ds(base + c * k, k)])
            return carry

        lax.fori_loop(0, nch, body, 0)

    return k_fn(table, idxc)


def _sc_gather2(p_tab, q_tab, dstc, srcc):
    d = p_tab.shape[1]
    dt = p_tab.dtype

    @functools.partial(
        pl.kernel,
        out_type=(jax.ShapeDtypeStruct((EPAD, d), dt),
                  jax.ShapeDtypeStruct((EPAD, d), dt)),
        mesh=_vmesh(),
        scratch_types=(
            [pltpu.VMEM((NCHG, KG), jnp.int32),
             pltpu.VMEM((NCHG, KG), jnp.int32)]
            + [pltpu.VMEM((KG, d), dt) for _ in range(2 * NB)]
            + [pltpu.SemaphoreType.DMA for _ in range(4 * NB)]
        ),
    )
    def k_fn(p_hbm, q_hbm, dst_hbm, src_hbm, pd_hbm, qs_hbm,
             idxd, idxs, *rest):
        buf_p = rest[0:NB]
        buf_q = rest[NB:2 * NB]
        gsem_p = rest[2 * NB:3 * NB]
        gsem_q = rest[3 * NB:4 * NB]
        wsem_p = rest[4 * NB:5 * NB]
        wsem_q = rest[5 * NB:6 * NB]
        wid = lax.axis_index("s") * NC + lax.axis_index("c")
        base = wid * EPT
        pltpu.sync_copy(dst_hbm.at[wid], idxd)
        pltpu.sync_copy(src_hbm.at[wid], idxs)

        def body(i, carry):
            hs = []
            for b in range(NB):
                c = i * NB + b

                @pl.when(i > 0)
                def _(b=b):
                    pltpu.make_async_copy(
                        pd_hbm.at[pl.ds(0, KG)], buf_p[b], wsem_p[b]).wait()
                    pltpu.make_async_copy(
                        qs_hbm.at[pl.ds(0, KG)], buf_q[b], wsem_q[b]).wait()

                hp = pltpu.async_copy(p_hbm.at[idxd.at[c]], buf_p[b],
                                      gsem_p[b])
                hq = pltpu.async_copy(q_hbm.at[idxs.at[c]], buf_q[b],
                                      gsem_q[b])
                hs.append((hp, hq))
            for b in range(NB):
                c = i * NB + b
                hp, hq = hs[b]
                hp.wait()
                pltpu.async_copy(buf_p[b],
                                 pd_hbm.at[pl.ds(base + c * KG, KG)],
                                 wsem_p[b])
                hq.wait()
                pltpu.async_copy(buf_q[b],
                                 qs_hbm.at[pl.ds(base + c * KG, KG)],
                                 wsem_q[b])
            return carry

        lax.fori_loop(0, NCHG // NB, body, 0)
        for b in range(NB):
            pltpu.make_async_copy(
                pd_hbm.at[pl.ds(0, KG)], buf_p[b], wsem_p[b]).wait()
            pltpu.make_async_copy(
                qs_hbm.at[pl.ds(0, KG)], buf_q[b], wsem_q[b]).wait()

    return k_fn(p_tab, q_tab, dstc, srcc)


def _sc_scatter(m, dstc, zrows):
    NBS = 2

    @functools.partial(
        pl.kernel,
        out_type=jax.ShapeDtypeStruct((NC, ACC_ROWS, HID), jnp.float32),
        mesh=_vmesh(),
        scratch_types=(
            [pltpu.VMEM((NCH, KE), jnp.int32)]
            + [pltpu.VMEM((KE, HID), jnp.float32) for _ in range(NBS)]
            + [pltpu.VMEM_SHARED((ACC_ROWS, HID), jnp.float32)]
            + [pltpu.SemaphoreType.DMA for _ in range(2 * NBS)]
        ),
    )
    def k_fn(m_hbm, dst_hbm, z_hbm, s_hbm, idxd, *rest):
        bufs = rest[0:NBS]
        acc = rest[NBS]
        rsem = rest[NBS + 1:2 * NBS + 1]
        asem = rest[2 * NBS + 1:3 * NBS + 1]
        cid = lax.axis_index("c")
        sid = lax.axis_index("s")
        wid = sid * NC + cid
        base = wid * EPT
        pltpu.sync_copy(z_hbm, acc.at[pl.ds(sid * RPT, RPT)])
        pltpu.sync_copy(dst_hbm.at[wid], idxd)
        plsc.subcore_barrier()

        def body(i, carry):
            hs = []
            for b in range(NBS):
                c = i * NBS + b

                @pl.when(i > 0)
                def _(b=b):
                    pltpu.make_async_copy(
                        m_hbm.at[pl.ds(0, KE)], bufs[b], asem[b]).wait()

                hs.append(pltpu.async_copy(
                    m_hbm.at[pl.ds(base + c * KE, KE)], bufs[b], rsem[b]))
            for b in range(NBS):
                c = i * NBS + b
                hs[b].wait()
                pltpu.async_copy(bufs[b], acc.at[idxd.at[c]], asem[b],
                                 add=True)
            return carry

        lax.fori_loop(0, NCH // NBS, body, 0)
        for b in range(NBS):
            pltpu.make_async_copy(
                m_hbm.at[pl.ds(0, KE)], bufs[b], asem[b]).wait()
        plsc.subcore_barrier()
        pltpu.sync_copy(acc.at[pl.ds(sid * RPT, RPT)],
                        s_hbm.at[cid, pl.ds(sid * RPT, RPT)])

    return k_fn(m, dstc, zrows)


def _layernorm(a, g, b):
    mu = jnp.mean(a, axis=-1, keepdims=True)
    var = jnp.mean((a - mu) ** 2, axis=-1, keepdims=True)
    return (a - mu) * lax.rsqrt(var + 1e-5) * g + b


def _dot(a, b, preferred_element_type=jnp.float32):
    return jnp.dot(a, b, preferred_element_type=preferred_element_type)


def _pack_bf16(pv):
    d2 = pv.shape[-1] // 2
    lo = pv[:, :d2].astype(jnp.bfloat16).astype(jnp.float32)
    hi = pv[:, d2:].astype(jnp.bfloat16).astype(jnp.float32)
    lo_u = lax.bitcast_convert_type(lo, jnp.uint32)
    hi_u = lax.bitcast_convert_type(hi, jnp.uint32)
    return lax.bitcast_convert_type(hi_u | (lo_u >> 16), jnp.int32)


def _unpack_bf16(w):
    u = lax.bitcast_convert_type(w, jnp.uint32)
    lo = lax.bitcast_convert_type(u << 16, jnp.float32)
    hi = lax.bitcast_convert_type(u & jnp.uint32(0xFFFF0000), jnp.float32)
    return lo, hi

def _row_spec(bn, d):
    return pl.BlockSpec((bn, d), lambda i: (i, 0))


def _w_spec(shape):
    if len(shape) == 1:
        return pl.BlockSpec(shape, lambda i: (0,))
    return pl.BlockSpec(shape, lambda i: (0, 0))


def _tc_embed_nodes(h0, xf, p):
    wl, bl = p["embed_x"]["lin"]["w"], p["embed_x"]["lin"]["b"]
    gx, bx = p["embed_x"]["ln"]["g"], p["embed_x"]["ln"]["b"]
    pf = p["embed_feat"]
    wf1, bf1 = pf["l1"]["w"], pf["l1"]["b"]
    wf2, bf2 = pf["l2"]["w"], pf["l2"]["b"]
    gf, bf = pf["ln"]["g"], pf["ln"]["b"]
    df = xf.shape[1]

    def body(h0_ref, xf_ref, wl_ref, bl_ref, gx_ref, bx_ref,
             wf1_ref, bf1_ref, wf2_ref, bf2_ref, gf_ref, bf_ref,
             hres_ref, pack_ref):
        h = jnp.maximum(h0_ref[...], 0.0)
        h = _dot(h, wl_ref[...], preferred_element_type=jnp.float32) + bl_ref[...]
        h = _layernorm(h, gx_ref[...], bx_ref[...])
        f = _dot(xf_ref[...], wf1_ref[...], preferred_element_type=jnp.float32) + bf1_ref[...]
        f = jnp.maximum(f, 0.0)
        f = _dot(f, wf2_ref[...], preferred_element_type=jnp.float32) + bf2_ref[...]
        f = _layernorm(f, gf_ref[...], bf_ref[...])
        hres_ref[...] = h
        pack_ref[...] = jnp.concatenate([h, f], axis=-1)

    return pl.pallas_call(
        body,
        grid=(N // BN_N,),
        in_specs=[_row_spec(BN_N, HID), _row_spec(BN_N, df),
                  _w_spec(wl.shape), _w_spec(bl.shape),
                  _w_spec(gx.shape), _w_spec(bx.shape),
                  _w_spec(wf1.shape), _w_spec(bf1.shape),
                  _w_spec(wf2.shape), _w_spec(bf2.shape),
                  _w_spec(gf.shape), _w_spec(bf.shape)],
        out_specs=(_row_spec(BN_N, HID), _row_spec(BN_N, 2 * HID)),
        out_shape=(jax.ShapeDtypeStruct((N, HID), jnp.float32),
                   jax.ShapeDtypeStruct((N, 2 * HID), jnp.float32)),
    )(h0, xf, wl, bl, gx, bx, wf1, bf1, wf2, bf2, gf, bf)


def _tc_embed_edges(ea, p):
    w1, b1 = p["l1"]["w"], p["l1"]["b"]
    w2, b2 = p["l2"]["w"], p["l2"]["b"]
    g, b = p["ln"]["g"], p["ln"]["b"]
    de = ea.shape[1]

    def body(ea_ref, w1_ref, b1_ref, w2_ref, b2_ref, g_ref, b_ref, out_ref):
        a = _dot(ea_ref[...], w1_ref[...], preferred_element_type=jnp.float32) + b1_ref[...]
        a = jnp.maximum(a, 0.0)
        a = _dot(a, w2_ref[...], preferred_element_type=jnp.float32) + b2_ref[...]
        out_ref[...] = _layernorm(a, g_ref[...], b_ref[...])

    return pl.pallas_call(
        body,
        grid=(EPAD // BN_E,),
        in_specs=[_row_spec(BN_E, de),
                  _w_spec(w1.shape), _w_spec(b1.shape),
                  _w_spec(w2.shape), _w_spec(b2.shape),
                  _w_spec(g.shape), _w_spec(b.shape)],
        out_specs=_row_spec(BN_E, HID),
        out_shape=jax.ShapeDtypeStruct((EPAD, HID), jnp.float32),
    )(ea, w1, b1, w2, b2, g, b)


def _tc_pq(h, wd, ws, relu_h):
    din = h.shape[1]

    def body(h_ref, wd_ref, ws_ref, p_ref, q_ref):
        hv = h_ref[...]
        if relu_h:
            hv = jnp.maximum(hv, 0.0)
        p_ref[...] = _pack_bf16(_dot(hv, wd_ref[...]))
        q_ref[...] = _pack_bf16(_dot(hv, ws_ref[...]))

    return pl.pallas_call(
        body,
        grid=(N // BN_N,),
        in_specs=[_row_spec(BN_N, din), _w_spec(wd.shape), _w_spec(ws.shape)],
        out_specs=(_row_spec(BN_N, HID), _row_spec(BN_N, HID)),
        out_shape=(jax.ShapeDtypeStruct((N, HID), jnp.int32),
                   jax.ShapeDtypeStruct((N, HID), jnp.int32)),
    )(h, wd, ws)


def _tc_edge_mlp(pd, qs, e, wc, b1, w2, b2, bn_e, relu_e):
    se = bn_e["g"] * (1.0 / jnp.sqrt(1.0 + 1e-5))
    be = bn_e["b"]

    def body(pd_ref, qs_ref, e_ref, wc_ref, b1_ref, w2_ref, b2_ref,
             se_ref, be_ref, m_ref, eo_ref):
        er = e_ref[...]
        if relu_e:
            er = jnp.maximum(er, 0.0)
        pd_lo, pd_hi = _unpack_bf16(pd_ref[...])
        qs_lo, qs_hi = _unpack_bf16(qs_ref[...])
        c = _dot(er, wc_ref[...], preferred_element_type=jnp.float32)
        b1v = b1_ref[...]
        z_lo = jnp.maximum(pd_lo + qs_lo + c[:, :HID] + b1v[:HID], 0.0)
        z_hi = jnp.maximum(pd_hi + qs_hi + c[:, HID:] + b1v[HID:], 0.0)
        m = (_dot(z_lo, w2_ref[:HID, :], preferred_element_type=jnp.float32)
             + _dot(z_hi, w2_ref[HID:, :], preferred_element_type=jnp.float32)
             + b2_ref[...])
        m_ref[...] = m
        eo_ref[...] = er + m * se_ref[...] + be_ref[...]

    return pl.pallas_call(
        body,
        grid=(EPAD // BN_E,),
        in_specs=[_row_spec(BN_E, HID), _row_spec(BN_E, HID),
                  _row_spec(BN_E, HID),
                  _w_spec(wc.shape), _w_spec(b1.shape),
                  _w_spec(w2.shape), _w_spec(b2.shape),
                  _w_spec(se.shape), _w_spec(be.shape)],
        out_specs=(_row_spec(BN_E, HID), _row_spec(BN_E, HID)),
        out_shape=(jax.ShapeDtypeStruct((EPAD, HID), jnp.float32),
                   jax.ShapeDtypeStruct((EPAD, HID), jnp.float32)),
    )(pd, qs, e, wc, b1, w2, b2, se, be)


def _tc_update(h_prev, s, bn_x, relu_h, lin=None):
    sx = bn_x["g"] * (1.0 / jnp.sqrt(1.0 + 1e-5))
    bx = bn_x["b"]
    args = [h_prev, s, sx, bx]
    in_specs = [_row_spec(BN_N, HID),
                pl.BlockSpec((NC, BN_N, HID), lambda i: (0, i, 0)),
                _w_spec(sx.shape), _w_spec(bx.shape)]
    if lin is not None:
        args += [lin["w"], lin["b"]]
        in_specs += [_w_spec(lin["w"].shape), _w_spec(lin["b"].shape)]

    def body(h_ref, s_ref, sx_ref, bx_ref, *rest):
        hv = h_ref[...]
        if relu_h:
            hv = jnp.maximum(hv, 0.0)
        ssum = s_ref[0, :, :] + s_ref[1, :, :]
        hs = hv + ssum * sx_ref[...] + bx_ref[...]
        if lin is not None:
            w_ref, b_ref, out_ref = rest
            out_ref[...] = _dot(hs, w_ref[...],
                                preferred_element_type=jnp.float32) + b_ref[...]
        else:
            out_ref, = rest
            out_ref[...] = hs

    dout = HID if lin is None else lin["w"].shape[1]
    return pl.pallas_call(
        body,
        grid=(N // BN_N,),
        in_specs=in_specs,
        out_specs=_row_spec(BN_N, dout),
        out_shape=jax.ShapeDtypeStruct((N, dout), jnp.float32),
    )(*args)


def kernel(x, edge_index, edge_attr, x_feat, params):
    x = x.astype(jnp.int32)
    src = edge_index[0].astype(jnp.int32)
    dst = edge_index[1].astype(jnp.int32)

    dst_g = jnp.pad(dst, (0, EPAD - E)).reshape(NT, NCHG, KG)
    src_g = jnp.pad(src, (0, EPAD - E)).reshape(NT, NCHG, KG)
    dst_s = jnp.pad(dst, (0, EPAD - E), constant_values=N).reshape(NT, NCH, KE)
    xc = jnp.pad(x, (0, XPAD - N)).reshape(NT, NCHX, KX)
    ea = jnp.pad(edge_attr, ((0, EPAD - E), (0, 0)))
    zrows = jnp.zeros((RPT, HID), jnp.float32)

    p = params
    h0 = _sc_gather1(p["embed_x"]["table"], xc)[:N]
    h_res, hcat = _tc_embed_nodes(h0, x_feat, p)
    e = _tc_embed_edges(ea, p["embed_adj"])

    h_in, relu_h = hcat, False
    out = None
    for li in range(4):
        cp = p["conv0"] if li == 0 else p["convs"][li - 1]
        d = h_in.shape[1]
        w1 = cp["m1"]["w"]
        wd = w1[:d] - w1[d:2 * d]
        ws = w1[d:2 * d]
        wc = w1[2 * d:]
        p_tab, q_tab = _tc_pq(h_in, wd, ws, relu_h)
        pd, qs = _sc_gather2(p_tab, q_tab, dst_g, src_g)
        m, e = _tc_edge_mlp(pd, qs, e, wc, cp["m1"]["b"], cp["m2"]["w"],
                            cp["m2"]["b"], cp["bn_e"], relu_e=(li > 0))
        s = _sc_scatter(m, dst_s, zrows)
        if li < 3:
            h_res = _tc_update(h_res, s, cp["bn_x"], relu_h=(li > 0))
            h_in, relu_h = h_res, True
        else:
            out = _tc_update(h_res, s, cp["bn_x"], relu_h=True,
                             lin=p["lin_out"])
    return out
